# Initial kernel scaffold; baseline (speedup 1.0000x reference)
import jax
import jax.numpy as jnp
from jax.experimental import pallas as pl


def kernel(x, edge_index, batch, W1, b1, W2, b2, W3, b3, g1, be1, g2, be2, g3, be3, fW1, fb1, fW2, fb2):
    raise NotImplementedError("write your pallas kernel here")


# speedup vs baseline: 15.6748x; 15.6748x over previous
import functools

import jax
import jax.numpy as jnp
from jax import lax
from jax.experimental import pallas as pl
from jax.experimental.pallas import tpu as pltpu
from jax.experimental.pallas import tpu_sc as plsc

N = 10000
E = 320000
D = 128
H = 128
C = 16
G = 64

NC = 2
NS = 16
NW = NC * NS
EPW = E // NW
K = 80
STEPS = EPW // K

RPS_A = 632
RPS_B = N - 15 * RPS_A

DEGW = 8


def _sc_mesh():
    return plsc.VectorSubcoreMesh(core_axis_name="c", subcore_axis_name="s")


def _each_subcore_rows(s, fn):
    @pl.when(s < NS - 1)
    def _():
        fn(s * RPS_A, RPS_A)

    @pl.when(s == NS - 1)
    def _():
        fn((NS - 1) * RPS_A, RPS_B)


def _sc_deg(dstm, zer, ones):
    @functools.partial(
        pl.kernel,
        mesh=_sc_mesh(),
        out_type=jax.ShapeDtypeStruct((NC, N, DEGW), jnp.float32),
        scratch_types=[
            pltpu.VMEM((STEPS, K), jnp.int32),
            pltpu.VMEM((K, DEGW), jnp.float32),
            pltpu.VMEM_SHARED((N, DEGW), jnp.float32),
        ],
    )
    def deg_kernel(dst_hbm, zer_hbm, ones_hbm, out_hbm, dst_v, ones_v, acc):
        c = lax.axis_index("c")
        s = lax.axis_index("s")
        wid = s * NC + c
        pltpu.sync_copy(dst_hbm.at[wid], dst_v)
        pltpu.sync_copy(ones_hbm, ones_v)
        _each_subcore_rows(s, lambda off, ln: pltpu.sync_copy(
            zer_hbm.at[pl.ds(off, ln)], acc.at[pl.ds(off, ln)]))
        plsc.subcore_barrier()

        def body(j, carry):
            pltpu.sync_copy(ones_v, acc.at[dst_v.at[j]], add=True)
            return carry

        lax.fori_loop(0, STEPS, body, 0)
        plsc.subcore_barrier()
        _each_subcore_rows(s, lambda off, ln: pltpu.sync_copy(
            acc.at[pl.ds(off, ln)], out_hbm.at[c, pl.ds(off, ln)]))

    return deg_kernel(dstm, zer, ones)


def _sc_scatter(hp, srcm, dstm, zer):
    @functools.partial(
        pl.kernel,
        mesh=_sc_mesh(),
        out_type=jax.ShapeDtypeStruct((NC, N, H), jnp.float32),
        scratch_types=[
            pltpu.VMEM((STEPS, K), jnp.int32),
            pltpu.VMEM((STEPS, K), jnp.int32),
            pltpu.VMEM((K, H), jnp.float32),
            pltpu.VMEM_SHARED((N, H), jnp.float32),
            pltpu.SemaphoreType.DMA,
        ],
    )
    def scat_kernel(hp_hbm, src_hbm, dst_hbm, zer_hbm, out_hbm,
                    src_v, dst_v, rows_v, acc, sem):
        c = lax.axis_index("c")
        s = lax.axis_index("s")
        wid = s * NC + c
        pltpu.sync_copy(src_hbm.at[wid], src_v)
        pltpu.sync_copy(dst_hbm.at[wid], dst_v)
        _each_subcore_rows(s, lambda off, ln: pltpu.sync_copy(
            zer_hbm.at[pl.ds(off, ln)], acc.at[pl.ds(off, ln)]))
        plsc.subcore_barrier()

        def body(j, carry):
            pltpu.async_copy(hp_hbm.at[src_v.at[j]], rows_v, sem).wait()
            pltpu.sync_copy(rows_v, acc.at[dst_v.at[j]], add=True)
            return carry

        lax.fori_loop(0, STEPS, body, 0)
        plsc.subcore_barrier()
        _each_subcore_rows(s, lambda off, ln: pltpu.sync_copy(
            acc.at[pl.ds(off, ln)], out_hbm.at[c, pl.ds(off, ln)]))

    return scat_kernel(hp, srcm, dstm, zer)


def _tc_prep(x, degp, W1):
    def body(x_ref, degp_ref, w_ref, hp_ref, dinv_ref):
        deg = degp_ref[0, :, 0:1] + degp_ref[1, :, 0:1] + 1.0
        dinv = lax.rsqrt(deg)
        xc = jnp.nan_to_num(x_ref[...], nan=0.0, posinf=1e6, neginf=-1e6)
        h = jnp.dot(xc, w_ref[...], preferred_element_type=jnp.float32)
        hp_ref[...] = dinv * h
        dinv_ref[...] = dinv

    return pl.pallas_call(
        body,
        out_shape=(jax.ShapeDtypeStruct((N, H), jnp.float32),
                   jax.ShapeDtypeStruct((N, 1), jnp.float32)),
    )(x, degp, W1)


def _bn_relu(pre, g, be):
    mu = jnp.mean(pre, axis=0, keepdims=True)
    cen = pre - mu
    var = jnp.mean(cen * cen, axis=0, keepdims=True)
    return jnp.maximum(cen * lax.rsqrt(var + 1e-5) * g + be, 0.0)


def _tc_mid(a, hp, dinv, b, g, be, Wn):
    def body(a_ref, hp_ref, dinv_ref, b_ref, g_ref, be_ref, w_ref, out_ref):
        dinv = dinv_ref[...]
        pre = dinv * (a_ref[0] + a_ref[1] + hp_ref[...]) + b_ref[...]
        h = _bn_relu(pre, g_ref[...], be_ref[...])
        out_ref[...] = dinv * jnp.dot(h, w_ref[...],
                                      preferred_element_type=jnp.float32)

    return pl.pallas_call(
        body,
        out_shape=jax.ShapeDtypeStruct((N, H), jnp.float32),
    )(a, hp, dinv, b, g, be, Wn)


def _tc_final(a, hp, dinv, b, g, be, batr, fW1, fb1, fW2, fb2):
    def body(a_ref, hp_ref, dinv_ref, b_ref, g_ref, be_ref, bat_ref,
             fw1_ref, fb1_ref, fw2_ref, fb2_ref, out_ref):
        dinv = dinv_ref[...]
        pre = dinv * (a_ref[0] + a_ref[1] + hp_ref[...]) + b_ref[...]
        h = _bn_relu(pre, g_ref[...], be_ref[...])
        seg = lax.broadcasted_iota(jnp.int32, (G, 1), 0)
        oh = (seg == bat_ref[...]).astype(jnp.float32)
        sums = jnp.dot(oh, h, preferred_element_type=jnp.float32)
        cnts = jnp.sum(oh, axis=1, keepdims=True)
        pooled = sums / jnp.maximum(cnts, 1.0)
        p1 = jnp.maximum(
            jnp.dot(pooled, fw1_ref[...], preferred_element_type=jnp.float32)
            + fb1_ref[...], 0.0)
        out_ref[...] = jnp.dot(p1, fw2_ref[...],
                               preferred_element_type=jnp.float32) + fb2_ref[...]

    return pl.pallas_call(
        body,
        out_shape=jax.ShapeDtypeStruct((G, C), jnp.float32),
    )(a, hp, dinv, b, g, be, batr, fW1, fb1, fW2, fb2)


def kernel(x, edge_index, batch, W1, b1, W2, b2, W3, b3,
           g1, be1, g2, be2, g3, be3, fW1, fb1, fW2, fb2):
    srcm = edge_index[0].reshape(NW, STEPS, K)
    dstm = edge_index[1].reshape(NW, STEPS, K)
    batr = batch.reshape(1, N)
    zer_nh = jnp.zeros((N, H), jnp.float32)
    zer_nd = jnp.zeros((N, DEGW), jnp.float32)
    ones_kd = jnp.ones((K, DEGW), jnp.float32)

    degp = _sc_deg(dstm, zer_nd, ones_kd)
    hp1, dinv = _tc_prep(x, degp, W1)
    a1 = _sc_scatter(hp1, srcm, dstm, zer_nh)
    hp2 = _tc_mid(a1, hp1, dinv, b1, g1, be1, W2)
    a2 = _sc_scatter(hp2, srcm, dstm, zer_nh)
    hp3 = _tc_mid(a2, hp2, dinv, b2, g2, be2, W3)
    a3 = _sc_scatter(hp3, srcm, dstm, zer_nh)
    return _tc_final(a3, hp3, dinv, b3, g3, be3, batr, fW1, fb1, fW2, fb2)


# speedup vs baseline: 15.7620x; 1.0056x over previous
import functools

import jax
import jax.numpy as jnp
from jax import lax
from jax.experimental import pallas as pl
from jax.experimental.pallas import tpu as pltpu
from jax.experimental.pallas import tpu_sc as plsc

N = 10000
E = 320000
D = 128
H = 128
C = 16
G = 64

NC = 2
NS = 16
NW = NC * NS
EPW = E // NW
K = 80
STEPS = EPW // K

EPWP = 10240
PAD = EPWP - EPW
SSTEPS = EPWP // K
SB = 16
NSUP = SSTEPS // SB

RPS_A = 632
RPS_B = N - 15 * RPS_A

DEGW = 128


def _sc_mesh():
    return plsc.VectorSubcoreMesh(core_axis_name="c", subcore_axis_name="s")


def _each_subcore_rows(s, fn):
    @pl.when(s < NS - 1)
    def _():
        fn(s * RPS_A, RPS_A)

    @pl.when(s == NS - 1)
    def _():
        fn((NS - 1) * RPS_A, RPS_B)


def _sc_deg(dstm, zer, ones):
    @functools.partial(
        pl.kernel,
        mesh=_sc_mesh(),
        out_type=jax.ShapeDtypeStruct((NC, N, DEGW), jnp.float32),
        scratch_types=[
            pltpu.VMEM((STEPS, K), jnp.int32),
            pltpu.VMEM((K, DEGW), jnp.float32),
            pltpu.VMEM_SHARED((N, DEGW), jnp.float32),
        ],
    )
    def deg_kernel(dst_hbm, zer_hbm, ones_hbm, out_hbm, dst_v, ones_v, acc):
        c = lax.axis_index("c")
        s = lax.axis_index("s")
        wid = s * NC + c
        pltpu.sync_copy(dst_hbm.at[wid], dst_v)
        pltpu.sync_copy(ones_hbm, ones_v)
        _each_subcore_rows(s, lambda off, ln: pltpu.sync_copy(
            zer_hbm.at[pl.ds(off, ln)], acc.at[pl.ds(off, ln)]))
        plsc.subcore_barrier()

        def body(j, carry):
            pltpu.sync_copy(ones_v, acc.at[dst_v.at[j]], add=True)
            return carry

        lax.fori_loop(0, STEPS, body, 0)
        plsc.subcore_barrier()
        _each_subcore_rows(s, lambda off, ln: pltpu.sync_copy(
            acc.at[pl.ds(off, ln)], out_hbm.at[c, pl.ds(off, ln)]))

    return deg_kernel(dstm, zer, ones)


def _sc_scatter(hp, srcp, dstp, zer):
    @functools.partial(
        pl.kernel,
        mesh=_sc_mesh(),
        out_type=jax.ShapeDtypeStruct((NC, N, H), jnp.float32),
        scratch_types=[
            pltpu.VMEM((SB, K), jnp.int32),
            pltpu.VMEM((SB, K), jnp.int32),
            pltpu.VMEM((K, H), jnp.float32),
            pltpu.VMEM((K, H), jnp.float32),
            pltpu.VMEM((K, H), jnp.float32),
            pltpu.VMEM((K, H), jnp.float32),
            pltpu.VMEM_SHARED((N + 8, H), jnp.float32),
            pltpu.SemaphoreType.DMA,
            pltpu.SemaphoreType.DMA,
            pltpu.SemaphoreType.DMA,
            pltpu.SemaphoreType.DMA,
        ],
    )
    def scat_kernel(hp_hbm, src_hbm, dst_hbm, zer_hbm, out_hbm,
                    sidx, didx, rows0, rows1, rows2, rows3, acc,
                    gsem0, gsem1, gsem2, gsem3):
        c = lax.axis_index("c")
        s = lax.axis_index("s")
        wid = s * NC + c
        rows = (rows0, rows1, rows2, rows3)
        gsems = (gsem0, gsem1, gsem2, gsem3)

        _each_subcore_rows(s, lambda off, ln: pltpu.sync_copy(
            zer_hbm.at[pl.ds(off, ln)], acc.at[pl.ds(off, ln)]))
        plsc.subcore_barrier()

        def block(bo, carry):
            base = wid * SSTEPS + bo * SB
            pltpu.sync_copy(src_hbm.at[pl.ds(base, SB)], sidx)
            pltpu.sync_copy(dst_hbm.at[pl.ds(base, SB)], didx)
            for q in range(SB // 4):
                cps = [pltpu.make_async_copy(
                           hp_hbm.at[sidx.at[4 * q + u]], rows[u], gsems[u])
                       for u in range(4)]
                for cp in cps:
                    cp.start()
                for cp in cps:
                    cp.wait()
                for u in range(4):
                    pltpu.sync_copy(rows[u], acc.at[didx.at[4 * q + u]],
                                    add=True)
            return carry

        lax.fori_loop(0, NSUP, block, 0)
        plsc.subcore_barrier()
        _each_subcore_rows(s, lambda off, ln: pltpu.sync_copy(
            acc.at[pl.ds(off, ln)], out_hbm.at[c, pl.ds(off, ln)]))

    return scat_kernel(hp, srcp, dstp, zer)


def _tc_prep(x, degp, W1):
    def body(x_ref, degp_ref, w_ref, hp_ref, dinv_ref):
        deg = degp_ref[0, :, 0:1] + degp_ref[1, :, 0:1] + 1.0
        dinv = lax.rsqrt(deg)
        xc = jnp.nan_to_num(x_ref[...], nan=0.0, posinf=1e6, neginf=-1e6)
        h = jnp.dot(xc, w_ref[...], preferred_element_type=jnp.float32)
        hp_ref[...] = dinv * h
        dinv_ref[...] = dinv

    return pl.pallas_call(
        body,
        out_shape=(jax.ShapeDtypeStruct((N, H), jnp.float32),
                   jax.ShapeDtypeStruct((N, 1), jnp.float32)),
    )(x, degp, W1)


def _bn_relu(pre, g, be):
    mu = jnp.mean(pre, axis=0, keepdims=True)
    cen = pre - mu
    var = jnp.mean(cen * cen, axis=0, keepdims=True)
    return jnp.maximum(cen * lax.rsqrt(var + 1e-5) * g + be, 0.0)


def _tc_mid(a, hp, dinv, b, g, be, Wn):
    def body(a_ref, hp_ref, dinv_ref, b_ref, g_ref, be_ref, w_ref, out_ref):
        dinv = dinv_ref[...]
        pre = dinv * (a_ref[0] + a_ref[1] + hp_ref[...]) + b_ref[...]
        h = _bn_relu(pre, g_ref[...], be_ref[...])
        out_ref[...] = dinv * jnp.dot(h, w_ref[...],
                                      preferred_element_type=jnp.float32)

    return pl.pallas_call(
        body,
        out_shape=jax.ShapeDtypeStruct((N, H), jnp.float32),
    )(a, hp, dinv, b, g, be, Wn)


def _tc_final(a, hp, dinv, b, g, be, batr, fW1, fb1, fW2, fb2):
    def body(a_ref, hp_ref, dinv_ref, b_ref, g_ref, be_ref, bat_ref,
             fw1_ref, fb1_ref, fw2_ref, fb2_ref, out_ref):
        dinv = dinv_ref[...]
        pre = dinv * (a_ref[0] + a_ref[1] + hp_ref[...]) + b_ref[...]
        h = _bn_relu(pre, g_ref[...], be_ref[...])
        seg = lax.broadcasted_iota(jnp.int32, (G, 1), 0)
        oh = (seg == bat_ref[...]).astype(jnp.float32)
        sums = jnp.dot(oh, h, preferred_element_type=jnp.float32)
        cnts = jnp.sum(oh, axis=1, keepdims=True)
        pooled = sums / jnp.maximum(cnts, 1.0)
        p1 = jnp.maximum(
            jnp.dot(pooled, fw1_ref[...], preferred_element_type=jnp.float32)
            + fb1_ref[...], 0.0)
        out_ref[...] = jnp.dot(p1, fw2_ref[...],
                               preferred_element_type=jnp.float32) + fb2_ref[...]

    return pl.pallas_call(
        body,
        out_shape=jax.ShapeDtypeStruct((G, C), jnp.float32),
    )(a, hp, dinv, b, g, be, batr, fW1, fb1, fW2, fb2)


def kernel(x, edge_index, batch, W1, b1, W2, b2, W3, b3,
           g1, be1, g2, be2, g3, be3, fW1, fb1, fW2, fb2):
    pad_s = jnp.broadcast_to((jnp.arange(PAD, dtype=jnp.int32) % 8)[None],
                             (NW, PAD))
    pad_d = pad_s + N
    srcp = jnp.concatenate([edge_index[0].reshape(NW, EPW), pad_s],
                           axis=1).reshape(NW * SSTEPS, K)
    dstp = jnp.concatenate([edge_index[1].reshape(NW, EPW), pad_d],
                           axis=1).reshape(NW * SSTEPS, K)
    dstm = edge_index[1].reshape(NW, STEPS, K)
    batr = batch.reshape(1, N)
    zer_nh = jnp.zeros((N, H), jnp.float32)
    zer_nd = jnp.zeros((N, DEGW), jnp.float32)
    ones_kd = jnp.ones((K, DEGW), jnp.float32)

    degp = _sc_deg(dstm, zer_nd, ones_kd)
    hp1, dinv = _tc_prep(x, degp, W1)
    a1 = _sc_scatter(hp1, srcp, dstp, zer_nh)
    hp2 = _tc_mid(a1, hp1, dinv, b1, g1, be1, W2)
    a2 = _sc_scatter(hp2, srcp, dstp, zer_nh)
    hp3 = _tc_mid(a2, hp2, dinv, b2, g2, be2, W3)
    a3 = _sc_scatter(hp3, srcp, dstp, zer_nh)
    return _tc_final(a3, hp3, dinv, b3, g3, be3, batr, fW1, fb1, fW2, fb2)


# speedup vs baseline: 19.0880x; 1.2110x over previous
import functools

import jax
import jax.numpy as jnp
from jax import lax
from jax.experimental import pallas as pl
from jax.experimental.pallas import tpu as pltpu
from jax.experimental.pallas import tpu_sc as plsc

N = 10000
E = 320000
D = 128
H = 128
C = 16
G = 64

NC = 2
NS = 16
NW = NC * NS
EPW = E // NW
K = 80
STEPS = EPW // K

EPWP = 10240
PAD = EPWP - EPW
SSTEPS = EPWP // K
SB = 16
NSUP = SSTEPS // SB

RPS_A = 632
RPS_B = N - 15 * RPS_A

DEGW = 128


def _sc_mesh():
    return plsc.VectorSubcoreMesh(core_axis_name="c", subcore_axis_name="s")


def _each_subcore_rows(s, fn):
    @pl.when(s < NS - 1)
    def _():
        fn(s * RPS_A, RPS_A)

    @pl.when(s == NS - 1)
    def _():
        fn((NS - 1) * RPS_A, RPS_B)


def _sc_deg(dstm, zer, ones):
    @functools.partial(
        pl.kernel,
        mesh=_sc_mesh(),
        out_type=jax.ShapeDtypeStruct((NC, N, DEGW), jnp.float32),
        scratch_types=[
            pltpu.VMEM((STEPS, K), jnp.int32),
            pltpu.VMEM((K, DEGW), jnp.float32),
            pltpu.VMEM_SHARED((N, DEGW), jnp.float32),
        ],
    )
    def deg_kernel(dst_hbm, zer_hbm, ones_hbm, out_hbm, dst_v, ones_v, acc):
        c = lax.axis_index("c")
        s = lax.axis_index("s")
        wid = s * NC + c
        pltpu.sync_copy(dst_hbm.at[wid], dst_v)
        pltpu.sync_copy(ones_hbm, ones_v)
        _each_subcore_rows(s, lambda off, ln: pltpu.sync_copy(
            zer_hbm.at[pl.ds(off, ln)], acc.at[pl.ds(off, ln)]))
        plsc.subcore_barrier()

        def body(j, carry):
            pltpu.sync_copy(ones_v, acc.at[dst_v.at[j]], add=True)
            return carry

        lax.fori_loop(0, STEPS, body, 0)
        plsc.subcore_barrier()
        _each_subcore_rows(s, lambda off, ln: pltpu.sync_copy(
            acc.at[pl.ds(off, ln)], out_hbm.at[c, pl.ds(off, ln)]))

    return deg_kernel(dstm, zer, ones)


def _sc_scatter(hp, srcp, dstp, zer):
    @functools.partial(
        pl.kernel,
        mesh=_sc_mesh(),
        out_type=jax.ShapeDtypeStruct((NC, N, H), jnp.float32),
        scratch_types=[
            pltpu.VMEM((SB, K), jnp.int32),
            pltpu.VMEM((SB, K), jnp.int32),
            pltpu.VMEM((K, H), jnp.float32),
            pltpu.VMEM((K, H), jnp.float32),
            pltpu.VMEM((K, H), jnp.float32),
            pltpu.VMEM((K, H), jnp.float32),
            pltpu.VMEM_SHARED((N + 8, H), jnp.float32),
            pltpu.SemaphoreType.DMA,
            pltpu.SemaphoreType.DMA,
            pltpu.SemaphoreType.DMA,
            pltpu.SemaphoreType.DMA,
        ],
    )
    def scat_kernel(hp_hbm, src_hbm, dst_hbm, zer_hbm, out_hbm,
                    sidx, didx, rows0, rows1, rows2, rows3, acc,
                    gsem0, gsem1, gsem2, gsem3):
        c = lax.axis_index("c")
        s = lax.axis_index("s")
        wid = s * NC + c
        rows = (rows0, rows1, rows2, rows3)
        gsems = (gsem0, gsem1, gsem2, gsem3)

        _each_subcore_rows(s, lambda off, ln: pltpu.sync_copy(
            zer_hbm.at[pl.ds(off, ln)], acc.at[pl.ds(off, ln)]))
        plsc.subcore_barrier()

        def block(bo, carry):
            base = wid * SSTEPS + bo * SB
            pltpu.sync_copy(src_hbm.at[pl.ds(base, SB)], sidx)
            pltpu.sync_copy(dst_hbm.at[pl.ds(base, SB)], didx)
            cps = [None] * SB
            for i in range(SB):
                if i == 0:
                    for a in range(3):
                        cps[a] = pltpu.make_async_copy(
                            hp_hbm.at[sidx.at[a]], rows[a % 4], gsems[a % 4])
                        cps[a].start()
                if i + 3 < SB:
                    cps[i + 3] = pltpu.make_async_copy(
                        hp_hbm.at[sidx.at[i + 3]],
                        rows[(i + 3) % 4], gsems[(i + 3) % 4])
                    cps[i + 3].start()
                cps[i].wait()
                pltpu.sync_copy(rows[i % 4], acc.at[didx.at[i]], add=True)
            return carry

        lax.fori_loop(0, NSUP, block, 0)
        plsc.subcore_barrier()
        _each_subcore_rows(s, lambda off, ln: pltpu.sync_copy(
            acc.at[pl.ds(off, ln)], out_hbm.at[c, pl.ds(off, ln)]))

    return scat_kernel(hp, srcp, dstp, zer)


def _tc_prep(x, degp, W1):
    def body(x_ref, degp_ref, w_ref, hp_ref, dinv_ref):
        deg = degp_ref[0, :, 0:1] + degp_ref[1, :, 0:1] + 1.0
        dinv = lax.rsqrt(deg)
        xc = jnp.nan_to_num(x_ref[...], nan=0.0, posinf=1e6, neginf=-1e6)
        h = jnp.dot(xc, w_ref[...], preferred_element_type=jnp.float32)
        hp_ref[...] = dinv * h
        dinv_ref[...] = dinv

    return pl.pallas_call(
        body,
        out_shape=(jax.ShapeDtypeStruct((N, H), jnp.float32),
                   jax.ShapeDtypeStruct((N, 1), jnp.float32)),
    )(x, degp, W1)


def _bn_relu(pre, g, be):
    mu = jnp.mean(pre, axis=0, keepdims=True)
    cen = pre - mu
    var = jnp.mean(cen * cen, axis=0, keepdims=True)
    return jnp.maximum(cen * lax.rsqrt(var + 1e-5) * g + be, 0.0)


def _tc_mid(a, hp, dinv, b, g, be, Wn):
    def body(a_ref, hp_ref, dinv_ref, b_ref, g_ref, be_ref, w_ref, out_ref):
        dinv = dinv_ref[...]
        pre = dinv * (a_ref[0] + a_ref[1] + hp_ref[...]) + b_ref[...]
        h = _bn_relu(pre, g_ref[...], be_ref[...])
        out_ref[...] = dinv * jnp.dot(h, w_ref[...],
                                      preferred_element_type=jnp.float32)

    return pl.pallas_call(
        body,
        out_shape=jax.ShapeDtypeStruct((N, H), jnp.float32),
    )(a, hp, dinv, b, g, be, Wn)


def _tc_final(a, hp, dinv, b, g, be, batr, fW1, fb1, fW2, fb2):
    def body(a_ref, hp_ref, dinv_ref, b_ref, g_ref, be_ref, bat_ref,
             fw1_ref, fb1_ref, fw2_ref, fb2_ref, out_ref):
        dinv = dinv_ref[...]
        pre = dinv * (a_ref[0] + a_ref[1] + hp_ref[...]) + b_ref[...]
        h = _bn_relu(pre, g_ref[...], be_ref[...])
        seg = lax.broadcasted_iota(jnp.int32, (G, 1), 0)
        oh = (seg == bat_ref[...]).astype(jnp.float32)
        sums = jnp.dot(oh, h, preferred_element_type=jnp.float32)
        cnts = jnp.sum(oh, axis=1, keepdims=True)
        pooled = sums / jnp.maximum(cnts, 1.0)
        p1 = jnp.maximum(
            jnp.dot(pooled, fw1_ref[...], preferred_element_type=jnp.float32)
            + fb1_ref[...], 0.0)
        out_ref[...] = jnp.dot(p1, fw2_ref[...],
                               preferred_element_type=jnp.float32) + fb2_ref[...]

    return pl.pallas_call(
        body,
        out_shape=jax.ShapeDtypeStruct((G, C), jnp.float32),
    )(a, hp, dinv, b, g, be, batr, fW1, fb1, fW2, fb2)


def kernel(x, edge_index, batch, W1, b1, W2, b2, W3, b3,
           g1, be1, g2, be2, g3, be3, fW1, fb1, fW2, fb2):
    pad_s = jnp.broadcast_to((jnp.arange(PAD, dtype=jnp.int32) % 8)[None],
                             (NW, PAD))
    pad_d = pad_s + N
    srcp = jnp.concatenate([edge_index[0].reshape(NW, EPW), pad_s],
                           axis=1).reshape(NW * SSTEPS, K)
    dstp = jnp.concatenate([edge_index[1].reshape(NW, EPW), pad_d],
                           axis=1).reshape(NW * SSTEPS, K)
    dstm = edge_index[1].reshape(NW, STEPS, K)
    batr = batch.reshape(1, N)
    zer_nh = jnp.zeros((N, H), jnp.float32)
    zer_nd = jnp.zeros((N, DEGW), jnp.float32)
    ones_kd = jnp.ones((K, DEGW), jnp.float32)

    degp = _sc_deg(dstm, zer_nd, ones_kd)
    hp1, dinv = _tc_prep(x, degp, W1)
    a1 = _sc_scatter(hp1, srcp, dstp, zer_nh)
    hp2 = _tc_mid(a1, hp1, dinv, b1, g1, be1, W2)
    a2 = _sc_scatter(hp2, srcp, dstp, zer_nh)
    hp3 = _tc_mid(a2, hp2, dinv, b2, g2, be2, W3)
    a3 = _sc_scatter(hp3, srcp, dstp, zer_nh)
    return _tc_final(a3, hp3, dinv, b3, g3, be3, batr, fW1, fb1, fW2, fb2)
